# baseline (speedup 1.0000x reference)
import jax
import jax.numpy as jnp
from jax import lax
from jax.experimental import pallas as pl
from jax.experimental.pallas import tpu as pltpu
from jax.experimental.pallas import tpu_sc as plsc

EMBED = 64
NUM_ROWS = 77
B_TOTAL = 16384 * 200
NUM_WORKERS = 32
PER_WORKER = B_TOTAL // NUM_WORKERS
STEP = 800
SPLIT = 400
NBUF = 2
STEPS = PER_WORKER // STEP
LANES = 16


def _embed_kernel(table_hbm, idx_hbm, out_hbm, tab_sh, tab_v,
                  idx_v, rows_v,
                  gsem, osem0, osem1, isem0, isem1):
    cid = lax.axis_index("c")
    sid = lax.axis_index("s")
    wid = sid * 2 + cid
    row_base = wid * PER_WORKER
    osems = [osem0, osem1]
    isems = [isem0, isem1]

    def idx_slice(i):
        return idx_hbm.at[pl.ds(pl.multiple_of(row_base + i * STEP, STEP), STEP)]

    def out_slice(i):
        return out_hbm.at[pl.ds(pl.multiple_of(row_base + i * STEP, STEP), STEP)]

    @pl.when(sid == 0)
    def _stage():
        pltpu.sync_copy(table_hbm, tab_sh)

    pltpu.sync_copy(table_hbm, tab_v)
    plsc.subcore_barrier()

    pltpu.async_copy(idx_slice(0), idx_v.at[0], isems[0])

    iota16 = lax.iota(jnp.int32, LANES)
    col_offs = [iota16 + c4 * LANES for c4 in range(EMBED // LANES)]
    lane_ids = [jnp.full((LANES,), u, jnp.int32) for u in range(LANES)]

    @pl.loop(0, STEPS, step=NBUF)
    def _outer(i0):
        for b in range(NBUF):
            i = i0 + b
            nb = (b + 1) % NBUF

            pltpu.make_async_copy(idx_slice(0), idx_v.at[b], isems[b]).wait()

            @pl.when(i + 1 < STEPS)
            def _prefetch():
                pltpu.async_copy(idx_slice(i + 1), idx_v.at[nb], isems[nb])

            @pl.when(i0 >= NBUF)
            def _reclaim():
                pltpu.make_async_copy(
                    rows_v.at[b], out_slice(0), osems[b]
                ).wait()

            stream = pltpu.async_copy(
                tab_sh.at[idx_v.at[b, pl.ds(0, SPLIT)]],
                rows_v.at[b, pl.ds(0, SPLIT)],
                gsem,
            )

            @plsc.parallel_loop(SPLIT, STEP, step=LANES, unroll=4)
            def _rows(r0):
                iv = idx_v[b, pl.ds(r0, LANES)]
                for u in range(LANES):
                    ubase = jnp.take_along_axis(iv, lane_ids[u], axis=0)
                    rvec = jnp.broadcast_to(r0 + u, (LANES,))
                    for c4 in range(EMBED // LANES):
                        vals = plsc.load_gather(
                            tab_v, [ubase, col_offs[c4]]
                        )
                        plsc.store_scatter(
                            rows_v.at[b], [rvec, col_offs[c4]], vals
                        )

            stream.wait()
            pltpu.async_copy(rows_v.at[b], out_slice(i), osems[b])

    for b in range(NBUF):
        pltpu.make_async_copy(rows_v.at[b], out_slice(0), osems[b]).wait()


@jax.jit
def kernel(day, table):
    idx1d = day.reshape(B_TOTAL).astype(jnp.int32)
    mesh = plsc.VectorSubcoreMesh(core_axis_name="c", subcore_axis_name="s")
    out = pl.kernel(
        _embed_kernel,
        mesh=mesh,
        compiler_params=pltpu.CompilerParams(
            use_tc_tiling_on_sc=False, needs_layout_passes=False
        ),
        out_type=jax.ShapeDtypeStruct((B_TOTAL, EMBED), jnp.float32),
        scratch_types=[
            pltpu.VMEM_SHARED((NUM_ROWS, EMBED), jnp.float32),
            pltpu.VMEM((NUM_ROWS, EMBED), jnp.float32),
            pltpu.VMEM((NBUF, STEP), jnp.int32),
            pltpu.VMEM((NBUF, STEP, EMBED), jnp.float32),
            pltpu.SemaphoreType.DMA,
            pltpu.SemaphoreType.DMA,
            pltpu.SemaphoreType.DMA,
            pltpu.SemaphoreType.DMA,
            pltpu.SemaphoreType.DMA,
        ],
    )(table, idx1d)
    return out.reshape(day.shape[0], day.shape[1], EMBED)


# speedup vs baseline: 2.7140x; 2.7140x over previous
import jax
import jax.numpy as jnp
from jax import lax
from jax.experimental import pallas as pl
from jax.experimental.pallas import tpu as pltpu
from jax.experimental.pallas import tpu_sc as plsc

EMBED = 64
NUM_ROWS = 77
BATCH = 16384
SEQ = 200
NUM_WORKERS = 32
BWIN = BATCH // NUM_WORKERS
NBH = BWIN // 128
TGRP = 10
NCELL = SEQ // TGRP
NBUF = 2
LANES = 16
CGRP = 16


def _embed_kernel(tabt_hbm, dayt_hbm, out_hbm, tab_v, idx_v, slab_v,
                  osem0, osem1, isem0, isem1):
    osems = [osem0, osem1]
    cid = lax.axis_index("c")
    sid = lax.axis_index("s")
    wid = sid * 2 + cid
    b0 = pl.multiple_of(wid * BWIN, BWIN)
    isems = [isem0, isem1]

    def idx_loads(g, slot, sem):
        t0 = g * TGRP
        for dt in range(TGRP):
            yield (
                dayt_hbm.at[
                    pl.ds(pl.multiple_of((t0 + dt) * BATCH + b0, BWIN), BWIN)
                ],
                idx_v.at[slot, pl.ds(dt * BWIN, BWIN)],
                sem,
            )

    pltpu.sync_copy(tabt_hbm, tab_v)

    for src, dst, sem in idx_loads(0, 0, isems[0]):
        pltpu.async_copy(src, dst, sem)

    iota16 = lax.iota(jnp.int32, LANES)
    col_ids = [jnp.full((LANES,), c, jnp.int32) for c in range(EMBED)]

    def drain_slab(sb):
        pltpu.make_async_copy(
            slab_v.at[0, 0], out_hbm.at[0, :, pl.ds(0, 128)], osems[sb]
        ).wait()

    @pl.loop(0, NCELL, step=NBUF)
    def _outer(g0):
        for bb in range(NBUF):
            g = g0 + bb
            nbb = (bb + 1) % NBUF

            for src, dst, sem in idx_loads(0, bb, isems[bb]):
                pltpu.make_async_copy(src, dst, sem).wait()

            @pl.when(g + 1 < NCELL)
            def _prefetch():
                for src, dst, sem in idx_loads(g + 1, nbb, isems[nbb]):
                    pltpu.async_copy(src, dst, sem)

            @pl.loop(0, TGRP, step=NBUF)
            def _per_t(dt0):
                for sb in range(NBUF):
                    dt = dt0 + sb
                    t = g * TGRP + dt

                    @pl.when(t >= NBUF)
                    def _reclaim():
                        for _ in range(NBH):
                            drain_slab(sb)

                    for bh in range(NBH):
                        @pl.loop(0, EMBED, step=CGRP)
                        def _cols(c0):
                            @plsc.parallel_loop(0, 128, step=LANES, unroll=2)
                            def _groups(v0):
                                iv = idx_v[
                                    bb,
                                    pl.ds(dt * BWIN + bh * 128 + v0, LANES),
                                ]
                                obase = v0 + iota16
                                for k in range(CGRP):
                                    c = c0 + k
                                    cvec = jnp.broadcast_to(c, (LANES,))
                                    vals = plsc.load_gather(
                                        tab_v, [iv + c * NUM_ROWS]
                                    )
                                    plsc.store_scatter(
                                        slab_v.at[sb, bh],
                                        [cvec, obase],
                                        vals,
                                    )

                        pltpu.async_copy(
                            slab_v.at[sb, bh],
                            out_hbm.at[
                                t, :,
                                pl.ds(pl.multiple_of(b0 + bh * 128, 128), 128)
                            ],
                            osems[sb],
                        )

    for sb in range(NBUF):
        for _ in range(NBH):
            drain_slab(sb)


@jax.jit
def kernel(day, table):
    dayt = day.astype(jnp.int32).T.reshape(SEQ * BATCH)
    tabt = table.T.reshape(EMBED * NUM_ROWS)
    mesh = plsc.VectorSubcoreMesh(core_axis_name="c", subcore_axis_name="s")
    out = pl.kernel(
        _embed_kernel,
        mesh=mesh,
        compiler_params=pltpu.CompilerParams(needs_layout_passes=False),
        out_type=jax.ShapeDtypeStruct((SEQ, EMBED, BATCH), jnp.float32),
        scratch_types=[
            pltpu.VMEM((EMBED * NUM_ROWS,), jnp.float32),
            pltpu.VMEM((NBUF, TGRP * BWIN), jnp.int32),
            pltpu.VMEM((NBUF, NBH, EMBED, 128), jnp.float32),
            pltpu.SemaphoreType.DMA,
            pltpu.SemaphoreType.DMA,
            pltpu.SemaphoreType.DMA,
            pltpu.SemaphoreType.DMA,
        ],
    )(tabt, dayt)
    return jnp.transpose(out, (2, 0, 1))
